# baseline (device time: 54437 ns/iter reference)
import jax
import jax.numpy as jnp
from jax import lax
from jax.experimental import pallas as pl
from jax.experimental.pallas import tpu as pltpu

N_DEV = 4
B, Sq, Skv, Dh = 2, 512, 512, 64
H_LOC = 8
D_LOC = H_LOC * Dh
D_MODEL = 768
HALF = D_MODEL // 2
CH = Sq // N_DEV
WINDOW = 128
N_STEP = 2 * (N_DEV - 1)

COMM_DT = jnp.bfloat16


def kernel(x, Wq, K_ext, V_ext, Wo):
    def body(x_ref, wq_ref, k_ref, v_ref, wo_ref, out_ref, part_ref,
             stage_r, recv_r, stage_l, recv_l,
             ssem_r, rsem_r, ssem_l, rsem_l):
        p = lax.axis_index("i")
        left = (p - 1) % N_DEV
        right = (p + 1) % N_DEV

        col0 = p * D_LOC
        wq_loc = wq_ref[:, pl.ds(col0, D_LOC)]
        wo_loc = wo_ref[pl.ds(col0, D_LOC), :]

        ki = lax.broadcasted_iota(jnp.int32, (CH, Skv), 1)

        def compute_chunk(c):
            r0 = c * CH
            qi = lax.broadcasted_iota(jnp.int32, (CH, Skv), 0) + r0
            mrow = jnp.where(jnp.abs(qi - ki) <= WINDOW,
                             jnp.float32(1.0), jnp.float32(0.0))
            for b in range(B):
                xq = x_ref[b, pl.ds(r0, CH), :]
                qc = jnp.dot(xq, wq_loc,
                             preferred_element_type=jnp.float32) * 0.125
                ctx_cols = []
                for h in range(H_LOC):
                    q = qc[:, h * Dh:(h + 1) * Dh]
                    k = k_ref[b, :, h, :]
                    v = v_ref[b, :, h, :]
                    s = lax.dot_general(
                        q, k, (((1,), (1,)), ((), ())),
                        preferred_element_type=jnp.float32)
                    w = jnp.exp(s) * mrow
                    denom = jnp.sum(w, axis=1, keepdims=True)
                    ctx_cols.append(
                        jnp.dot(w, v, preferred_element_type=jnp.float32)
                        / denom)
                ctx_b = jnp.concatenate(ctx_cols, axis=1)
                part_ref[b, pl.ds(r0, CH), :] = jnp.dot(
                    ctx_b, wo_loc, preferred_element_type=jnp.float32)

        def part_chunk(c, lo):
            return part_ref[:, pl.ds((c % N_DEV) * CH, CH), lo:lo + HALF]

        def put_out(c, lo, val):
            out_ref[:, pl.ds((c % N_DEV) * CH, CH), lo:lo + HALF] = (
                val.astype(jnp.float32))

        def start(src, dst, ssem, rsem, s, dest):
            rdma = pltpu.make_async_remote_copy(
                src_ref=src.at[s], dst_ref=dst.at[s],
                send_sem=ssem.at[s], recv_sem=rsem.at[s],
                device_id=(dest,), device_id_type=pl.DeviceIdType.MESH,
            )
            rdma.start()
            return rdma

        def start_both(s):
            return (start(stage_r, recv_r, ssem_r, rsem_r, s, right),
                    start(stage_l, recv_l, ssem_l, rsem_l, s, left))

        compute_chunk(p)
        stage_r[0] = part_chunk(p, 0).astype(COMM_DT)
        stage_l[0] = part_chunk(p, HALF).astype(COMM_DT)

        barrier_sem = pltpu.get_barrier_semaphore()
        for nbr in [left, right]:
            pl.semaphore_signal(
                barrier_sem, inc=1,
                device_id=(nbr,), device_id_type=pl.DeviceIdType.MESH,
            )
        pl.semaphore_wait(barrier_sem, 2)

        rdmas = []
        rdmas.extend(start_both(0))
        compute_chunk((p + 1) % N_DEV)
        compute_chunk((p + 3) % N_DEV)

        for s in range(N_DEV - 1):
            rr, rl = rdmas[-2], rdmas[-1]
            rr.wait_recv()
            rl.wait_recv()
            acc_r = recv_r[s].astype(jnp.float32) + part_chunk(p - s - 1, 0)
            acc_l = recv_l[s].astype(jnp.float32) + part_chunk(p + s + 1, HALF)
            if s < N_DEV - 2:
                stage_r[s + 1] = acc_r.astype(COMM_DT)
                stage_l[s + 1] = acc_l.astype(COMM_DT)
                rdmas.extend(start_both(s + 1))
                if s == 0:
                    compute_chunk((p + 2) % N_DEV)
            else:
                stage_r[N_DEV - 1] = acc_r.astype(COMM_DT)
                stage_l[N_DEV - 1] = acc_l.astype(COMM_DT)
                rdmas.extend(start_both(N_DEV - 1))
                put_out(p + 1, 0, acc_r)
                put_out(p - 1, HALF, acc_l)

        for t in range(N_DEV - 1):
            s = (N_DEV - 1) + t
            rr, rl = rdmas[-2], rdmas[-1]
            rr.wait_recv()
            rl.wait_recv()
            if t < N_DEV - 2:
                stage_r[s + 1] = recv_r[s]
                stage_l[s + 1] = recv_l[s]
                rdmas.extend(start_both(s + 1))
            put_out(p - t, 0, recv_r[s])
            put_out(p + t, HALF, recv_l[s])

        for rdma in rdmas:
            rdma.wait_send()

    chunk = (B, CH, HALF)
    return pl.pallas_call(
        body,
        out_shape=jax.ShapeDtypeStruct((B, Sq, D_MODEL), jnp.float32),
        in_specs=[pl.BlockSpec(memory_space=pltpu.VMEM)] * 5,
        out_specs=pl.BlockSpec(memory_space=pltpu.VMEM),
        scratch_shapes=[
            pltpu.VMEM((B, Sq, D_MODEL), jnp.float32),
            pltpu.VMEM((N_STEP,) + chunk, COMM_DT),
            pltpu.VMEM((N_STEP,) + chunk, COMM_DT),
            pltpu.VMEM((N_STEP,) + chunk, COMM_DT),
            pltpu.VMEM((N_STEP,) + chunk, COMM_DT),
            pltpu.SemaphoreType.DMA((N_STEP,)),
            pltpu.SemaphoreType.DMA((N_STEP,)),
            pltpu.SemaphoreType.DMA((N_STEP,)),
            pltpu.SemaphoreType.DMA((N_STEP,)),
        ],
        compiler_params=pltpu.CompilerParams(collective_id=0),
    )(x, Wq, K_ext, V_ext, Wo)


# device time: 51911 ns/iter; 1.0487x vs baseline; 1.0487x over previous
import jax
import jax.numpy as jnp
from jax import lax
from jax.experimental import pallas as pl
from jax.experimental.pallas import tpu as pltpu

N_DEV = 4
B, Sq, Skv, Dh = 2, 512, 512, 64
H_LOC = 8
D_LOC = H_LOC * Dh
D_MODEL = 768
HALF = D_MODEL // 2
CH = Sq // N_DEV
WINDOW = 128
N_STEP = 2 * (N_DEV - 1)

COMM_DT = jnp.bfloat16
MXU_DT = jnp.bfloat16


def kernel(x, Wq, K_ext, V_ext, Wo):
    def body(x_ref, wq_ref, k_ref, v_ref, wo_ref, out_ref, part_ref,
             stage_r, recv_r, stage_l, recv_l,
             ssem_r, rsem_r, ssem_l, rsem_l):
        p = lax.axis_index("i")
        left = (p - 1) % N_DEV
        right = (p + 1) % N_DEV

        col0 = p * D_LOC
        wq_loc = wq_ref[:, pl.ds(col0, D_LOC)].astype(MXU_DT)
        wo_loc = wo_ref[pl.ds(col0, D_LOC), :].astype(MXU_DT)

        qi = lax.broadcasted_iota(jnp.int32, (Sq, Skv), 0)
        ki = lax.broadcasted_iota(jnp.int32, (Sq, Skv), 1)
        mask01 = jnp.where(jnp.abs(qi - ki) <= WINDOW,
                           jnp.float32(1.0), jnp.float32(0.0))

        x2 = x_ref[...].reshape(B * Sq, D_MODEL).astype(MXU_DT)
        q_all = (jnp.dot(x2, wq_loc, preferred_element_type=jnp.float32)
                 * 0.125).astype(MXU_DT)

        ctx_rows = []
        for b in range(B):
            ctx_cols = []
            for h in range(H_LOC):
                q = q_all[b * Sq:(b + 1) * Sq, h * Dh:(h + 1) * Dh]
                k = k_ref[b, :, h, :].astype(MXU_DT)
                v = v_ref[b, :, h, :].astype(MXU_DT)
                s = lax.dot_general(
                    q, k, (((1,), (1,)), ((), ())),
                    preferred_element_type=jnp.float32)
                w = jnp.exp(s) * mask01
                denom = jnp.sum(w, axis=1, keepdims=True)
                ctx_cols.append(
                    jnp.dot(w.astype(MXU_DT), v,
                            preferred_element_type=jnp.float32) / denom)
            ctx_rows.append(jnp.concatenate(ctx_cols, axis=1))
        ctx_all = jnp.concatenate(ctx_rows, axis=0).astype(MXU_DT)
        part_ref[...] = jnp.dot(
            ctx_all, wo_loc,
            preferred_element_type=jnp.float32).reshape(B, Sq, D_MODEL)

        def part_chunk(c, lo):
            return part_ref[:, pl.ds((c % N_DEV) * CH, CH), lo:lo + HALF]

        def put_out(c, lo, val):
            out_ref[:, pl.ds((c % N_DEV) * CH, CH), lo:lo + HALF] = (
                val.astype(jnp.float32))

        def start(src, dst, ssem, rsem, s, dest):
            rdma = pltpu.make_async_remote_copy(
                src_ref=src.at[s], dst_ref=dst.at[s],
                send_sem=ssem.at[s], recv_sem=rsem.at[s],
                device_id=(dest,), device_id_type=pl.DeviceIdType.MESH,
            )
            rdma.start()
            return rdma

        def start_both(s):
            return (start(stage_r, recv_r, ssem_r, rsem_r, s, right),
                    start(stage_l, recv_l, ssem_l, rsem_l, s, left))

        stage_r[0] = part_chunk(p, 0).astype(COMM_DT)
        stage_l[0] = part_chunk(p, HALF).astype(COMM_DT)

        barrier_sem = pltpu.get_barrier_semaphore()
        for nbr in [left, right]:
            pl.semaphore_signal(
                barrier_sem, inc=1,
                device_id=(nbr,), device_id_type=pl.DeviceIdType.MESH,
            )
        pl.semaphore_wait(barrier_sem, 2)

        rdmas = list(start_both(0))

        for s in range(N_DEV - 1):
            rr, rl = rdmas[-2], rdmas[-1]
            rr.wait_recv()
            rl.wait_recv()
            acc_r = recv_r[s].astype(jnp.float32) + part_chunk(p - s - 1, 0)
            acc_l = recv_l[s].astype(jnp.float32) + part_chunk(p + s + 1, HALF)
            stage_r[s + 1] = acc_r.astype(COMM_DT)
            stage_l[s + 1] = acc_l.astype(COMM_DT)
            rdmas.extend(start_both(s + 1))
            if s == N_DEV - 2:
                put_out(p + 1, 0, acc_r)
                put_out(p - 1, HALF, acc_l)

        for t in range(N_DEV - 1):
            s = (N_DEV - 1) + t
            rr, rl = rdmas[-2], rdmas[-1]
            rr.wait_recv()
            rl.wait_recv()
            if t < N_DEV - 2:
                stage_r[s + 1] = recv_r[s]
                stage_l[s + 1] = recv_l[s]
                rdmas.extend(start_both(s + 1))
            put_out(p - t, 0, recv_r[s])
            put_out(p + t, HALF, recv_l[s])

        for rdma in rdmas:
            rdma.wait_send()

    chunk = (B, CH, HALF)
    return pl.pallas_call(
        body,
        out_shape=jax.ShapeDtypeStruct((B, Sq, D_MODEL), jnp.float32),
        in_specs=[pl.BlockSpec(memory_space=pltpu.VMEM)] * 5,
        out_specs=pl.BlockSpec(memory_space=pltpu.VMEM),
        scratch_shapes=[
            pltpu.VMEM((B, Sq, D_MODEL), jnp.float32),
            pltpu.VMEM((N_STEP + 1,) + chunk, COMM_DT),
            pltpu.VMEM((N_STEP + 1,) + chunk, COMM_DT),
            pltpu.VMEM((N_STEP + 1,) + chunk, COMM_DT),
            pltpu.VMEM((N_STEP + 1,) + chunk, COMM_DT),
            pltpu.SemaphoreType.DMA((N_STEP + 1,)),
            pltpu.SemaphoreType.DMA((N_STEP + 1,)),
            pltpu.SemaphoreType.DMA((N_STEP + 1,)),
            pltpu.SemaphoreType.DMA((N_STEP + 1,)),
        ],
        compiler_params=pltpu.CompilerParams(collective_id=0),
    )(x, Wq, K_ext, V_ext, Wo)


# device time: 47546 ns/iter; 1.1449x vs baseline; 1.0918x over previous
import jax
import jax.numpy as jnp
from jax import lax
from jax.experimental import pallas as pl
from jax.experimental.pallas import tpu as pltpu

N_DEV = 4
B, Sq, Skv, Dh = 2, 512, 512, 64
H_LOC = 8
D_LOC = H_LOC * Dh
D_MODEL = 768
HALF = D_MODEL // 2
CH = Sq // N_DEV
WINDOW = 128
N_STEP = 2 * (N_DEV - 1)

COMM_DT = jnp.bfloat16
MXU_DT = jnp.float32


def kernel(x, Wq, K_ext, V_ext, Wo):
    def body(x_ref, wq_ref, k_ref, v_ref, wo_ref, out_ref, part_ref,
             stage_r, recv_r, stage_l, recv_l,
             ssem_r, rsem_r, ssem_l, rsem_l):
        p = lax.axis_index("i")
        left = (p - 1) % N_DEV
        right = (p + 1) % N_DEV

        col0 = p * D_LOC
        wq_loc = wq_ref[:, pl.ds(col0, D_LOC)].astype(MXU_DT)
        wo_loc = wo_ref[pl.ds(col0, D_LOC), :].astype(MXU_DT)

        qi = lax.broadcasted_iota(jnp.int32, (Sq, Skv), 0)
        ki = lax.broadcasted_iota(jnp.int32, (Sq, Skv), 1)
        mask01 = jnp.where(jnp.abs(qi - ki) <= WINDOW,
                           jnp.float32(1.0), jnp.float32(0.0))

        x2 = x_ref[...].reshape(B * Sq, D_MODEL).astype(MXU_DT)
        q_all = (jnp.dot(x2, wq_loc, preferred_element_type=jnp.float32)
                 * 0.125).astype(MXU_DT)

        ctx_rows = []
        for b in range(B):
            ctx_cols = []
            for h in range(H_LOC):
                q = q_all[b * Sq:(b + 1) * Sq, h * Dh:(h + 1) * Dh]
                k = k_ref[b, :, h, :].astype(MXU_DT)
                v = v_ref[b, :, h, :].astype(MXU_DT)
                s = lax.dot_general(
                    q, k, (((1,), (1,)), ((), ())),
                    preferred_element_type=jnp.float32)
                w = jnp.exp(s) * mask01
                denom = jnp.sum(w, axis=1, keepdims=True)
                ctx_cols.append(
                    jnp.dot(w.astype(MXU_DT), v,
                            preferred_element_type=jnp.float32) / denom)
            ctx_rows.append(jnp.concatenate(ctx_cols, axis=1))
        ctx_all = jnp.concatenate(ctx_rows, axis=0).astype(MXU_DT)
        part_ref[...] = jnp.dot(
            ctx_all, wo_loc,
            preferred_element_type=jnp.float32).reshape(B, Sq, D_MODEL)

        def part_chunk(c, lo):
            return part_ref[:, pl.ds((c % N_DEV) * CH, CH), lo:lo + HALF]

        def put_out(c, lo, val):
            out_ref[:, pl.ds((c % N_DEV) * CH, CH), lo:lo + HALF] = (
                val.astype(jnp.float32))

        def start(src, dst, ssem, rsem, s, dest):
            rdma = pltpu.make_async_remote_copy(
                src_ref=src.at[s], dst_ref=dst.at[s],
                send_sem=ssem.at[s], recv_sem=rsem.at[s],
                device_id=(dest,), device_id_type=pl.DeviceIdType.MESH,
            )
            rdma.start()
            return rdma

        def start_both(s):
            return (start(stage_r, recv_r, ssem_r, rsem_r, s, right),
                    start(stage_l, recv_l, ssem_l, rsem_l, s, left))

        stage_r[0] = part_chunk(p, 0).astype(COMM_DT)
        stage_l[0] = part_chunk(p, HALF).astype(COMM_DT)

        barrier_sem = pltpu.get_barrier_semaphore()
        for nbr in [left, right]:
            pl.semaphore_signal(
                barrier_sem, inc=1,
                device_id=(nbr,), device_id_type=pl.DeviceIdType.MESH,
            )
        pl.semaphore_wait(barrier_sem, 2)

        rdmas = list(start_both(0))

        for s in range(N_DEV - 1):
            rr, rl = rdmas[-2], rdmas[-1]
            rr.wait_recv()
            rl.wait_recv()
            acc_r = recv_r[s].astype(jnp.float32) + part_chunk(p - s - 1, 0)
            acc_l = recv_l[s].astype(jnp.float32) + part_chunk(p + s + 1, HALF)
            stage_r[s + 1] = acc_r.astype(COMM_DT)
            stage_l[s + 1] = acc_l.astype(COMM_DT)
            rdmas.extend(start_both(s + 1))
            if s == N_DEV - 2:
                put_out(p + 1, 0, acc_r)
                put_out(p - 1, HALF, acc_l)

        for t in range(N_DEV - 1):
            s = (N_DEV - 1) + t
            rr, rl = rdmas[-2], rdmas[-1]
            rr.wait_recv()
            rl.wait_recv()
            if t < N_DEV - 2:
                stage_r[s + 1] = recv_r[s]
                stage_l[s + 1] = recv_l[s]
                rdmas.extend(start_both(s + 1))
            put_out(p - t, 0, recv_r[s])
            put_out(p + t, HALF, recv_l[s])

        for rdma in rdmas:
            rdma.wait_send()

    chunk = (B, CH, HALF)
    return pl.pallas_call(
        body,
        out_shape=jax.ShapeDtypeStruct((B, Sq, D_MODEL), jnp.float32),
        in_specs=[pl.BlockSpec(memory_space=pltpu.VMEM)] * 5,
        out_specs=pl.BlockSpec(memory_space=pltpu.VMEM),
        scratch_shapes=[
            pltpu.VMEM((B, Sq, D_MODEL), jnp.float32),
            pltpu.VMEM((N_STEP + 1,) + chunk, COMM_DT),
            pltpu.VMEM((N_STEP + 1,) + chunk, COMM_DT),
            pltpu.VMEM((N_STEP + 1,) + chunk, COMM_DT),
            pltpu.VMEM((N_STEP + 1,) + chunk, COMM_DT),
            pltpu.SemaphoreType.DMA((N_STEP + 1,)),
            pltpu.SemaphoreType.DMA((N_STEP + 1,)),
            pltpu.SemaphoreType.DMA((N_STEP + 1,)),
            pltpu.SemaphoreType.DMA((N_STEP + 1,)),
        ],
        compiler_params=pltpu.CompilerParams(collective_id=0),
    )(x, Wq, K_ext, V_ext, Wo)


# device time: 45761 ns/iter; 1.1896x vs baseline; 1.0390x over previous
import jax
import jax.numpy as jnp
from jax import lax
from jax.experimental import pallas as pl
from jax.experimental.pallas import tpu as pltpu

N_DEV = 4
B, Sq, Skv, Dh = 2, 512, 512, 64
H_LOC = 8
D_LOC = H_LOC * Dh
D_MODEL = 768
CH = Sq // N_DEV
WINDOW = 128

COMM_DT = jnp.bfloat16


def kernel(x, Wq, K_ext, V_ext, Wo):
    def body(x_ref, wq_ref, k_ref, v_ref, wo_ref, out_ref, part_ref,
             rs_stage, rs_recv, ag_stage, ag_recv,
             rs_ssem, rs_rsem, ag_ssem, ag_rsem):
        p = lax.axis_index("i")

        col0 = p * D_LOC
        wq_loc = wq_ref[:, pl.ds(col0, D_LOC)]
        wo_loc = wo_ref[pl.ds(col0, D_LOC), :]

        qi = lax.broadcasted_iota(jnp.int32, (Sq, Skv), 0)
        ki = lax.broadcasted_iota(jnp.int32, (Sq, Skv), 1)
        mask01 = jnp.where(jnp.abs(qi - ki) <= WINDOW,
                           jnp.float32(1.0), jnp.float32(0.0))

        x2 = x_ref[...].reshape(B * Sq, D_MODEL)
        q_all = jnp.dot(x2, wq_loc,
                        preferred_element_type=jnp.float32) * 0.125

        ctx_rows = []
        for b in range(B):
            ctx_cols = []
            for h in range(H_LOC):
                q = q_all[b * Sq:(b + 1) * Sq, h * Dh:(h + 1) * Dh]
                k = k_ref[b, :, h, :]
                v = v_ref[b, :, h, :]
                s = lax.dot_general(
                    q, k, (((1,), (1,)), ((), ())),
                    preferred_element_type=jnp.float32)
                w = jnp.exp(s) * mask01
                denom = jnp.sum(w, axis=1, keepdims=True)
                ctx_cols.append(
                    jnp.dot(w, v, preferred_element_type=jnp.float32)
                    / denom)
            ctx_rows.append(jnp.concatenate(ctx_cols, axis=1))
        ctx_all = jnp.concatenate(ctx_rows, axis=0)
        part_ref[...] = jnp.dot(
            ctx_all, wo_loc,
            preferred_element_type=jnp.float32).reshape(B, Sq, D_MODEL)

        def part_chunk(c):
            return part_ref[:, pl.ds((c % N_DEV) * CH, CH), :]

        def put_out(c, val):
            out_ref[:, pl.ds((c % N_DEV) * CH, CH), :] = (
                val.astype(jnp.float32))

        def start(src, dst, ssem, rsem, dest):
            rdma = pltpu.make_async_remote_copy(
                src_ref=src, dst_ref=dst, send_sem=ssem, recv_sem=rsem,
                device_id=(dest,), device_id_type=pl.DeviceIdType.MESH,
            )
            rdma.start()
            return rdma

        for d in range(1, N_DEV):
            rs_stage[d - 1] = part_chunk(p + d).astype(COMM_DT)

        barrier_sem = pltpu.get_barrier_semaphore()
        for d in range(1, N_DEV):
            pl.semaphore_signal(
                barrier_sem, inc=1,
                device_id=((p + d) % N_DEV,),
                device_id_type=pl.DeviceIdType.MESH,
            )
        pl.semaphore_wait(barrier_sem, N_DEV - 1)

        rdmas = []
        for d in range(1, N_DEV):
            rdmas.append(start(
                rs_stage.at[d - 1], rs_recv.at[d - 1],
                rs_ssem.at[d - 1], rs_rsem.at[d - 1], (p + d) % N_DEV))
        for r in rdmas:
            r.wait_recv()

        acc = part_chunk(p)
        for d in range(1, N_DEV):
            acc = acc + rs_recv[d - 1].astype(jnp.float32)
        put_out(p, acc)
        ag_stage[0] = acc.astype(COMM_DT)

        for d in range(1, N_DEV):
            rdmas.append(start(
                ag_stage.at[0], ag_recv.at[d - 1],
                ag_ssem.at[d - 1], ag_rsem.at[d - 1], (p + d) % N_DEV))
        for r in rdmas[N_DEV - 1:]:
            r.wait_recv()
        for d in range(1, N_DEV):
            put_out(p - d, ag_recv[d - 1])

        for r in rdmas:
            r.wait_send()

    chunk = (B, CH, D_MODEL)
    return pl.pallas_call(
        body,
        out_shape=jax.ShapeDtypeStruct((B, Sq, D_MODEL), jnp.float32),
        in_specs=[pl.BlockSpec(memory_space=pltpu.VMEM)] * 5,
        out_specs=pl.BlockSpec(memory_space=pltpu.VMEM),
        scratch_shapes=[
            pltpu.VMEM((B, Sq, D_MODEL), jnp.float32),
            pltpu.VMEM((N_DEV - 1,) + chunk, COMM_DT),
            pltpu.VMEM((N_DEV - 1,) + chunk, COMM_DT),
            pltpu.VMEM((1,) + chunk, COMM_DT),
            pltpu.VMEM((N_DEV - 1,) + chunk, COMM_DT),
            pltpu.SemaphoreType.DMA((N_DEV - 1,)),
            pltpu.SemaphoreType.DMA((N_DEV - 1,)),
            pltpu.SemaphoreType.DMA((N_DEV - 1,)),
            pltpu.SemaphoreType.DMA((N_DEV - 1,)),
        ],
        compiler_params=pltpu.CompilerParams(collective_id=0),
    )(x, Wq, K_ext, V_ext, Wo)
